# baseline (speedup 1.0000x reference)
import functools

import jax
import jax.numpy as jnp
from jax.experimental import pallas as pl
from jax.experimental.pallas import tpu as pltpu

_NUM_EXPERTS = 64
_TOP_K = 8
_ALPHA = 0.01

_NIBBLE_GROUP = 14


def _loss_body(x_ref, idx_ref, p_ref, c_ref, cnt_ref, l2_ref):
    x = x_ref[0]
    m = jnp.max(x, axis=0, keepdims=True)
    e = jnp.exp(x - m)
    ones = jnp.ones((1, x.shape[0]), jnp.float32)
    s = jax.lax.dot_general(
        ones, e, (((1,), (0,)), ((), ())),
        preferred_element_type=jnp.float32)
    r = 1.0 / s
    part = jax.lax.dot_general(
        e, r, (((1,), (1,)), ((), ())),
        preferred_element_type=jnp.float32)
    p_ref[0] = part

    cnt_ref[...] = jnp.zeros_like(cnt_ref)
    l2_ref[...] = jnp.zeros_like(l2_ref)

    idx = idx_ref[0]
    nvec = idx.shape[1] // 128
    hvals = jnp.arange(8, dtype=jnp.int32)
    for g0 in range(0, nvec, _NIBBLE_GROUP):
        g1 = min(g0 + _NIBBLE_GROUP, nvec)
        accs = [jnp.zeros((8, 128), jnp.int32) for _ in range(8)]
        for i in range(g0, g1):
            blk = idx[:, i * 128:(i + 1) * 128]
            hi = blk >> 3
            lo = blk & 7
            pw = jnp.left_shift(jnp.int32(1), lo << 2)
            for h in range(8):
                accs[h] = accs[h] + jnp.where(hi == hvals[h], pw, 0)
        for h in range(8):
            l2_ref[h, 0] = l2_ref[h, 0] + (accs[h] & 0x0F0F0F0F)
            l2_ref[h, 1] = l2_ref[h, 1] + ((accs[h] >> 4) & 0x0F0F0F0F)

    for h in range(8):
        for par in range(2):
            w = l2_ref[h, par]
            for b4 in range(4):
                lo_val = 2 * b4 + par
                cnt8 = (w >> (8 * b4)) & 0xFF
                row = 8 * h + lo_val
                cnt_ref[row:row + 1, :] = (
                    cnt_ref[row:row + 1, :]
                    + jnp.sum(cnt8, axis=0, keepdims=True))

    c_ref[0] = jnp.sum(cnt_ref[...].astype(jnp.float32), axis=1,
                       keepdims=True)


@jax.jit
def kernel(router_logits, expert_indices):
    batch, seq, ne = router_logits.shape
    num_tokens = batch * seq
    xt = jnp.transpose(router_logits, (0, 2, 1))
    it = jnp.transpose(expert_indices, (0, 2, 1))

    scale = _ALPHA * _NUM_EXPERTS * _NUM_EXPERTS / (
        _TOP_K * float(num_tokens) * float(num_tokens))

    psums, csums = pl.pallas_call(
        _loss_body,
        grid=(batch,),
        in_specs=[
            pl.BlockSpec((1, ne, seq), lambda b: (b, 0, 0)),
            pl.BlockSpec((1, _TOP_K, seq), lambda b: (b, 0, 0)),
        ],
        out_specs=[
            pl.BlockSpec((1, ne, 1), lambda b: (b, 0, 0)),
            pl.BlockSpec((1, ne, 1), lambda b: (b, 0, 0)),
        ],
        out_shape=[
            jax.ShapeDtypeStruct((batch, ne, 1), jnp.float32),
            jax.ShapeDtypeStruct((batch, ne, 1), jnp.float32),
        ],
        scratch_shapes=[
            pltpu.VMEM((_NUM_EXPERTS, 128), jnp.int32),
            pltpu.VMEM((8, 2, 8, 128), jnp.int32),
        ],
        compiler_params=pltpu.CompilerParams(
            dimension_semantics=("parallel",),
        ),
    )(xt, it)
    psum = jnp.sum(psums, axis=0)
    csum = jnp.sum(csums, axis=0)
    return (jnp.sum(psum * csum) * scale).reshape(())


# speedup vs baseline: 1.5086x; 1.5086x over previous
import functools

import jax
import jax.numpy as jnp
from jax.experimental import pallas as pl
from jax.experimental.pallas import tpu as pltpu

_NUM_EXPERTS = 64
_TOP_K = 8
_ALPHA = 0.01

_NIBBLE_GROUP = 14


def _loss_body(scale, x_ref, idx_ref, o_ref, acc_ref, cnt_ref, l2_ref):
    nb = pl.num_programs(0)
    ns = pl.num_programs(1)
    b = pl.program_id(0)
    sblk = pl.program_id(1)
    first = jnp.logical_and(b == 0, sblk == 0)
    last = jnp.logical_and(b == nb - 1, sblk == ns - 1)

    x = x_ref[0]
    m = jnp.max(x, axis=0, keepdims=True)
    e = jnp.exp(x - m)
    ones = jnp.ones((1, x.shape[0]), jnp.float32)
    s = jax.lax.dot_general(
        ones, e, (((1,), (0,)), ((), ())),
        preferred_element_type=jnp.float32)
    r = 1.0 / s
    part = jax.lax.dot_general(
        e, r, (((1,), (1,)), ((), ())),
        preferred_element_type=jnp.float32)

    @pl.when(first)
    def _():
        acc_ref[...] = jnp.zeros_like(acc_ref)
        cnt_ref[...] = jnp.zeros_like(cnt_ref)

    acc_ref[:, 0:1] = acc_ref[:, 0:1] + part

    l2_ref[...] = jnp.zeros_like(l2_ref)

    idx = idx_ref[0]
    nvec = idx.shape[1] // 128
    hvals = jnp.arange(8, dtype=jnp.int32)
    for g0 in range(0, nvec, _NIBBLE_GROUP):
        g1 = min(g0 + _NIBBLE_GROUP, nvec)
        accs = [jnp.zeros((8, 128), jnp.int32) for _ in range(8)]
        for i in range(g0, g1):
            blk = idx[:, i * 128:(i + 1) * 128]
            hi = blk >> 3
            lo = blk & 7
            pw = jnp.left_shift(jnp.int32(1), lo << 2)
            for h in range(8):
                accs[h] = accs[h] + jnp.where(hi == hvals[h], pw, 0)
        for h in range(8):
            l2_ref[h, 0] = l2_ref[h, 0] + (accs[h] & 0x0F0F0F0F)
            l2_ref[h, 1] = l2_ref[h, 1] + ((accs[h] >> 4) & 0x0F0F0F0F)

    for h in range(8):
        for par in range(2):
            w = l2_ref[h, par]
            for b4 in range(4):
                lo_val = 2 * b4 + par
                cnt8 = (w >> (8 * b4)) & 0xFF
                row = 8 * h + lo_val
                cnt_ref[row:row + 1, :] = (
                    cnt_ref[row:row + 1, :]
                    + jnp.sum(cnt8, axis=0, keepdims=True))

    @pl.when(last)
    def _():
        psum = jnp.sum(acc_ref[...], axis=1, keepdims=True)
        csum = jnp.sum(cnt_ref[...].astype(jnp.float32), axis=1,
                       keepdims=True)
        o_ref[...] = jnp.sum(psum * csum, axis=0, keepdims=True) * scale


@jax.jit
def kernel(router_logits, expert_indices):
    batch, seq, ne = router_logits.shape
    num_tokens = batch * seq
    xt = jnp.transpose(router_logits, (0, 2, 1))
    it = jnp.transpose(expert_indices, (0, 2, 1))

    scale = _ALPHA * _NUM_EXPERTS * _NUM_EXPERTS / (
        _TOP_K * float(num_tokens) * float(num_tokens))

    n_sblk = 1
    sb = seq // n_sblk
    out = pl.pallas_call(
        functools.partial(_loss_body, scale),
        grid=(batch, n_sblk),
        in_specs=[
            pl.BlockSpec((1, ne, sb), lambda b, s: (b, 0, s)),
            pl.BlockSpec((1, _TOP_K, sb), lambda b, s: (b, 0, s)),
        ],
        out_specs=pl.BlockSpec((1, 1), lambda b, s: (0, 0)),
        out_shape=jax.ShapeDtypeStruct((1, 1), jnp.float32),
        scratch_shapes=[
            pltpu.VMEM((_NUM_EXPERTS, 128), jnp.float32),
            pltpu.VMEM((_NUM_EXPERTS, 128), jnp.int32),
            pltpu.VMEM((8, 2, 8, 128), jnp.int32),
        ],
        compiler_params=pltpu.CompilerParams(
            dimension_semantics=("arbitrary", "arbitrary"),
        ),
    )(xt, it)
    return out.reshape(())
